# baseline (speedup 1.0000x reference)
import functools

import jax
import jax.numpy as jnp
from jax.experimental import pallas as pl
from jax.experimental.pallas import tpu as pltpu

EPS = 1e-5


def _fps_kernel(pc_ref, fidx_ref, newxyz_ref, cn_ref, *, npoint, n):
    b = pc_ref.shape[0]
    X = pc_ref[:, 0, :]
    Y = pc_ref[:, 1, :]
    Z = pc_ref[:, 2, :]
    lane = jax.lax.broadcasted_iota(jnp.int32, (b, n), 1)
    subl = jax.lax.broadcasted_iota(jnp.int32, (b, n), 0)
    subl1 = jax.lax.broadcasted_iota(jnp.int32, (b, 1), 0)
    sublp = jax.lax.broadcasted_iota(jnp.int32, (b, npoint), 0)
    col = jax.lax.broadcasted_iota(jnp.int32, (b, npoint), 1)

    def body(i, carry):
        dists, far, aidx, ax, ay, az, an = carry
        sel = col == i
        aidx = jnp.where(sel, jnp.broadcast_to(far, aidx.shape), aidx)
        mask = lane == far
        cx = jnp.sum(jnp.where(mask, X, 0.0), axis=1, keepdims=True)
        cy = jnp.sum(jnp.where(mask, Y, 0.0), axis=1, keepdims=True)
        cz = jnp.sum(jnp.where(mask, Z, 0.0), axis=1, keepdims=True)
        cn = cx * cx + cy * cy + cz * cz
        ax = jnp.where(sel, jnp.broadcast_to(cx, ax.shape), ax)
        ay = jnp.where(sel, jnp.broadcast_to(cy, ay.shape), ay)
        az = jnp.where(sel, jnp.broadcast_to(cz, az.shape), az)
        an = jnp.where(sel, jnp.broadcast_to(cn, an.shape), an)
        d = (X - cx) ** 2 + (Y - cy) ** 2 + (Z - cz) ** 2
        dists = jnp.minimum(dists, d)
        far = jnp.argmax(dists, axis=1, keepdims=True).astype(jnp.int32)
        return dists, far, aidx, ax, ay, az, an

    dists0 = jnp.maximum(subl.astype(jnp.float32), 1e10)
    far0 = jnp.minimum(subl1, 0)
    aidx0 = jnp.minimum(sublp, 0)
    az0 = aidx0.astype(jnp.float32)
    _, _, aidx, ax, ay, az, an = jax.lax.fori_loop(
        0, npoint, body, (dists0, far0, aidx0, az0, az0, az0, az0))
    fidx_ref[...] = aidx
    newxyz_ref[:, 0, :] = ax
    newxyz_ref[:, 1, :] = ay
    newxyz_ref[:, 2, :] = az
    cn_ref[...] = an


def _fps(pc, npoint):
    b, _, n = pc.shape
    hb = b // 2
    return pl.pallas_call(
        functools.partial(_fps_kernel, npoint=npoint, n=n),
        grid=(2,),
        in_specs=[pl.BlockSpec((hb, 3, n), lambda i: (i, 0, 0))],
        out_specs=[pl.BlockSpec((hb, npoint), lambda i: (i, 0)),
                   pl.BlockSpec((hb, 3, npoint), lambda i: (i, 0, 0)),
                   pl.BlockSpec((hb, npoint), lambda i: (i, 0))],
        out_shape=[jax.ShapeDtypeStruct((b, npoint), jnp.int32),
                   jax.ShapeDtypeStruct((b, 3, npoint), jnp.float32),
                   jax.ShapeDtypeStruct((b, npoint), jnp.float32)],
        compiler_params=pltpu.CompilerParams(
            dimension_semantics=("parallel",)),
    )(pc)


def _sq_dists(C, P):
    ones3 = jnp.zeros((3, 1), dtype=jnp.float32) + 1.0
    pn = jnp.sum(P * P, axis=0, keepdims=True)
    cn = jax.lax.dot_general(C * C, ones3, (((0,), (0,)), ((), ())))
    cp = jax.lax.dot_general(C, P, (((0,), (0,)), ((), ())))
    return (cn + pn) - 2.0 * cp


def _knn_mlp_pool(sq, F, beta, W2, k):
    s, n = sq.shape
    lane = jax.lax.broadcasted_iota(jnp.int32, (s, n), 1)
    acc = None
    for j in range(k):
        first = jnp.argmin(sq, axis=1, keepdims=True).astype(jnp.int32)
        one = lane == first
        maskf = one.astype(jnp.float32)
        G = jax.lax.dot_general(maskf, F, (((1,), (1,)), ((), ())))
        Hh = jax.nn.relu(G + beta)
        O = jax.lax.dot_general(Hh, W2, (((1,), (1,)), ((), ())),
                                precision=jax.lax.Precision.HIGHEST)
        acc = O if acc is None else jnp.maximum(acc, O)
        if j + 1 < k:
            sq = jnp.where(one, jnp.inf, sq)
    return acc


def _sa1_kernel(pc_ref, cxyz_ref, Wp_ref, A1_ref, b1_ref, W2_ref, out_ref,
                *, k):
    P = pc_ref[0]
    C = cxyz_ref[0]
    sq = _sq_dists(C, P)
    F = jax.lax.dot_general(Wp_ref[...], P, (((1,), (0,)), ((), ())),
                            precision=jax.lax.Precision.HIGHEST)
    beta = b1_ref[...] - jax.lax.dot_general(
        C, A1_ref[...], (((0,), (1,)), ((), ())),
        precision=jax.lax.Precision.HIGHEST)
    out_ref[0] = _knn_mlp_pool(sq, F, beta, W2_ref[...], k)


def _sa2_kernel(pc_ref, cxyz_ref, feat_ref, A1_ref, A2_ref, b1_ref, W2_ref,
                out_ref, *, k):
    P = pc_ref[0]
    C = cxyz_ref[0]
    feat = feat_ref[0]
    sq = _sq_dists(C, P)
    F = (jax.lax.dot_general(A1_ref[...], P, (((1,), (0,)), ((), ())),
                             precision=jax.lax.Precision.HIGHEST)
         + jax.lax.dot_general(A2_ref[...], feat, (((1,), (1,)), ((), ())),
                               precision=jax.lax.Precision.HIGHEST))
    beta = b1_ref[...] - jax.lax.dot_general(
        C, A1_ref[...], (((0,), (1,)), ((), ())),
        precision=jax.lax.Precision.HIGHEST)
    out_ref[0] = _knn_mlp_pool(sq, F, beta, W2_ref[...], k)


def _sa1(pc, cxyz, Wp, A1, b1, W2, k):
    b, _, n = pc.shape
    s = cxyz.shape[2]
    h2 = W2.shape[0]
    return pl.pallas_call(
        functools.partial(_sa1_kernel, k=k),
        grid=(b,),
        in_specs=[
            pl.BlockSpec((1, 3, n), lambda i: (i, 0, 0)),
            pl.BlockSpec((1, 3, s), lambda i: (i, 0, 0)),
            pl.BlockSpec(Wp.shape, lambda i: (0, 0)),
            pl.BlockSpec(A1.shape, lambda i: (0, 0)),
            pl.BlockSpec(b1.shape, lambda i: (0, 0)),
            pl.BlockSpec(W2.shape, lambda i: (0, 0)),
        ],
        out_specs=pl.BlockSpec((1, s, h2), lambda i: (i, 0, 0)),
        out_shape=jax.ShapeDtypeStruct((b, s, h2), jnp.float32),
        compiler_params=pltpu.CompilerParams(
            dimension_semantics=("parallel",)),
    )(pc, cxyz, Wp, A1, b1, W2)


def _sa2(pc, cxyz, feat, A1, A2, b1, W2, k):
    b, _, n = pc.shape
    s = cxyz.shape[2]
    cf = feat.shape[2]
    h2 = W2.shape[0]
    return pl.pallas_call(
        functools.partial(_sa2_kernel, k=k),
        grid=(b,),
        in_specs=[
            pl.BlockSpec((1, 3, n), lambda i: (i, 0, 0)),
            pl.BlockSpec((1, 3, s), lambda i: (i, 0, 0)),
            pl.BlockSpec((1, n, cf), lambda i: (i, 0, 0)),
            pl.BlockSpec(A1.shape, lambda i: (0, 0)),
            pl.BlockSpec(A2.shape, lambda i: (0, 0)),
            pl.BlockSpec(b1.shape, lambda i: (0, 0)),
            pl.BlockSpec(W2.shape, lambda i: (0, 0)),
        ],
        out_specs=pl.BlockSpec((1, s, h2), lambda i: (i, 0, 0)),
        out_shape=jax.ShapeDtypeStruct((b, s, h2), jnp.float32),
        compiler_params=pltpu.CompilerParams(
            dimension_semantics=("parallel",)),
    )(pc, cxyz, feat, A1, A2, b1, W2)


def _tail_kernel(xyz_ref, feat_ref, W3x_ref, W3f_ref, b3a_ref, W3b_ref,
                 b3b_ref, Wf1_ref, bn1_ref, W1s_ref, Wf2_ref, bn2_ref,
                 W2s_ref, logit_ref):
    xyz = xyz_ref[0]
    feat = feat_ref[0]
    h = jax.nn.relu(
        jax.lax.dot_general(W3x_ref[...], xyz, (((1,), (0,)), ((), ())),
                            precision=jax.lax.Precision.HIGHEST)
        + jax.lax.dot_general(W3f_ref[...], feat, (((1,), (1,)), ((), ())),
                              precision=jax.lax.Precision.HIGHEST)
        + b3a_ref[...])
    o = jax.lax.dot_general(W3b_ref[...], h, (((1,), (0,)), ((), ())),
                            precision=jax.lax.Precision.HIGHEST)
    o = o + b3b_ref[...]
    x = jnp.max(o, axis=1, keepdims=True)
    rs = jnp.sqrt(1.0 + EPS)
    y = jax.lax.dot_general(Wf1_ref[...], x, (((1,), (0,)), ((), ())),
                            precision=jax.lax.Precision.HIGHEST)
    y = W1s_ref[...] * (y + bn1_ref[..., 0:1]) / rs + bn1_ref[..., 1:2]
    y = jax.nn.relu(y)
    z = jax.lax.dot_general(Wf2_ref[...], y, (((1,), (0,)), ((), ())),
                            precision=jax.lax.Precision.HIGHEST)
    z = W2s_ref[...] * (z + bn2_ref[..., 0:1]) / rs + bn2_ref[..., 1:2]
    logit_ref[0] = jax.nn.relu(z)


def _tail(xyz, feat, W3a, b3a, W3b, b3b, Wf1, bf1, gf1, bef1, Wf2, bf2, gf2,
          bef2):
    b, _, s = xyz.shape
    W3x = W3a[:, :3]
    W3f = W3a[:, 3:]
    b3ac = b3a[:, None]
    b3bc = b3b[:, None]
    bn1 = jnp.stack([bf1, bef1], axis=1)
    bn2 = jnp.stack([bf2, bef2], axis=1)
    g1c = gf1[:, None]
    g2c = gf2[:, None]
    full = lambda a: pl.BlockSpec(a.shape, lambda i: tuple(0 for _ in a.shape))
    out = pl.pallas_call(
        _tail_kernel,
        grid=(b,),
        in_specs=[
            pl.BlockSpec((1, 3, s), lambda i: (i, 0, 0)),
            pl.BlockSpec((1, s, feat.shape[2]), lambda i: (i, 0, 0)),
            full(W3x), full(W3f), full(b3ac), full(W3b), full(b3bc),
            full(Wf1), full(bn1), full(g1c), full(Wf2), full(bn2), full(g2c),
        ],
        out_specs=pl.BlockSpec((1, 128, 1), lambda i: (i, 0, 0)),
        out_shape=jax.ShapeDtypeStruct((b, 128, 1), jnp.float32),
        compiler_params=pltpu.CompilerParams(
            dimension_semantics=("parallel",)),
    )(xyz, feat, W3x, W3f, b3ac, W3b, b3bc, Wf1, bn1, g1c, Wf2, bn2, g2c)
    return out[:, :, 0]


def kernel(point_cloud, W1a, b1a, W1b, b1b, W2a, b2a, W2b, b2b, W3a, b3a,
           W3b, b3b, Wf1, bf1, gf1, bef1, Wf2, bf2, gf2, bef2):
    pc = point_cloud
    _, l1_xyz, _ = _fps(pc, 512)
    Wp1 = W1a[:, :3] + W1a[:, 3:]
    p1 = _sa1(pc, l1_xyz, Wp1, W1a[:, :3], b1a[None, :], W1b, 12)
    _, l2_xyz, _ = _fps(l1_xyz, 128)
    p2 = _sa2(l1_xyz, l2_xyz, p1, W2a[:, :3], W2a[:, 3:],
              b2a[None, :], W2b, 8)
    logit = _tail(l2_xyz, p2, W3a, b3a, W3b, b3b,
                  Wf1, bf1, gf1, bef1, Wf2, bf2, gf2, bef2)
    l21_points = jnp.transpose(p2, (0, 2, 1))
    return (logit, l21_points)


# speedup vs baseline: 1.7446x; 1.7446x over previous
import functools

import jax
import jax.numpy as jnp
from jax.experimental import pallas as pl
from jax.experimental.pallas import tpu as pltpu
from jax.experimental.pallas import tpu_sc as plsc

EPS = 1e-5


def _fps_kernel(pc_ref, fidx_ref, newxyz_ref, cn_ref, *, npoint, n):
    b = pc_ref.shape[0]
    X = pc_ref[:, 0, :]
    Y = pc_ref[:, 1, :]
    Z = pc_ref[:, 2, :]
    lane = jax.lax.broadcasted_iota(jnp.int32, (b, n), 1)
    subl = jax.lax.broadcasted_iota(jnp.int32, (b, n), 0)
    subl1 = jax.lax.broadcasted_iota(jnp.int32, (b, 1), 0)
    sublp = jax.lax.broadcasted_iota(jnp.int32, (b, npoint), 0)
    col = jax.lax.broadcasted_iota(jnp.int32, (b, npoint), 1)

    def body(i, carry):
        dists, far, aidx, ax, ay, az, an = carry
        sel = col == i
        aidx = jnp.where(sel, jnp.broadcast_to(far, aidx.shape), aidx)
        mask = lane == far
        cx = jnp.sum(jnp.where(mask, X, 0.0), axis=1, keepdims=True)
        cy = jnp.sum(jnp.where(mask, Y, 0.0), axis=1, keepdims=True)
        cz = jnp.sum(jnp.where(mask, Z, 0.0), axis=1, keepdims=True)
        cn = cx * cx + cy * cy + cz * cz
        ax = jnp.where(sel, jnp.broadcast_to(cx, ax.shape), ax)
        ay = jnp.where(sel, jnp.broadcast_to(cy, ay.shape), ay)
        az = jnp.where(sel, jnp.broadcast_to(cz, az.shape), az)
        an = jnp.where(sel, jnp.broadcast_to(cn, an.shape), an)
        d = (X - cx) ** 2 + (Y - cy) ** 2 + (Z - cz) ** 2
        dists = jnp.minimum(dists, d)
        far = jnp.argmax(dists, axis=1, keepdims=True).astype(jnp.int32)
        return dists, far, aidx, ax, ay, az, an

    dists0 = jnp.maximum(subl.astype(jnp.float32), 1e10)
    far0 = jnp.minimum(subl1, 0)
    aidx0 = jnp.minimum(sublp, 0)
    az0 = aidx0.astype(jnp.float32)
    _, _, aidx, ax, ay, az, an = jax.lax.fori_loop(
        0, npoint, body, (dists0, far0, aidx0, az0, az0, az0, az0))
    fidx_ref[...] = aidx
    newxyz_ref[:, 0, :] = ax
    newxyz_ref[:, 1, :] = ay
    newxyz_ref[:, 2, :] = az
    cn_ref[...] = an


def _fps(pc, npoint):
    b, _, n = pc.shape
    return pl.pallas_call(
        functools.partial(_fps_kernel, npoint=npoint, n=n),
        out_shape=[jax.ShapeDtypeStruct((b, npoint), jnp.int32),
                   jax.ShapeDtypeStruct((b, 3, npoint), jnp.float32),
                   jax.ShapeDtypeStruct((b, npoint), jnp.float32)],
    )(pc)


def _sq_dists_t(Pt, C, cn_row):
    x, y, z = Pt[:, 0:1], Pt[:, 1:2], Pt[:, 2:3]
    pn = (x * x + y * y) + z * z
    cp = jax.lax.dot_general(Pt, C, (((1,), (0,)), ((), ())))
    return (cn_row + pn) - 2.0 * cp


def _knn_mlp_pool_t(sq, F, beta, W2, k):
    n, s = sq.shape
    rown = jax.lax.broadcasted_iota(jnp.int32, (n, s), 0)
    acc = None
    for j in range(k):
        first = jnp.argmin(sq, axis=0, keepdims=True).astype(jnp.int32)
        one = rown == first
        maskf = one.astype(jnp.float32)
        G = jax.lax.dot_general(maskf, F, (((0,), (1,)), ((), ())))
        Hh = jax.nn.relu(G + beta)
        O = jax.lax.dot_general(Hh, W2, (((1,), (1,)), ((), ())),
                                precision=jax.lax.Precision.HIGHEST)
        acc = O if acc is None else jnp.maximum(acc, O)
        if j + 1 < k:
            sq = jnp.where(one, jnp.inf, sq)
    return acc


def _sel1_kernel(pt_ref, cxyz_ref, cn_ref, Wp_ref, A1_ref, b1_ref, idx_ref,
                 ft_ref, beta_ref, *, k, n):
    Pt = pt_ref[0]
    C = cxyz_ref[0]
    sq = _sq_dists_t(Pt, C, cn_ref[0])
    s = sq.shape[1]
    F = jax.lax.dot_general(Pt, Wp_ref[...], (((1,), (1,)), ((), ())),
                            precision=jax.lax.Precision.HIGHEST)
    h1 = F.shape[1]
    ft_ref[0, :, 0:h1] = F
    ft_ref[0, :, h1:] = jnp.zeros_like(F)
    beta_ref[0] = b1_ref[...] - jax.lax.dot_general(
        C, A1_ref[...], (((0,), (1,)), ((), ())),
        precision=jax.lax.Precision.HIGHEST)
    rown = jax.lax.broadcasted_iota(jnp.int32, (n, s), 0)
    off = pl.program_id(0) * n
    for j in range(k):
        first = jnp.argmin(sq, axis=0, keepdims=True).astype(jnp.int32)
        idx_ref[0, j:j + 1, :] = first + off
        if j + 1 < k:
            one = rown == first
            sq = jnp.where(one, jnp.inf, sq)


def _sel1(pt, cxyz, cn, Wp, A1, b1, k):
    b, n, _ = pt.shape
    s = cxyz.shape[2]
    h1 = Wp.shape[0]
    return pl.pallas_call(
        functools.partial(_sel1_kernel, k=k, n=n),
        grid=(b,),
        in_specs=[
            pl.BlockSpec((1, n, 3), lambda i: (i, 0, 0)),
            pl.BlockSpec((1, 3, s), lambda i: (i, 0, 0)),
            pl.BlockSpec((1, 1, s), lambda i: (i, 0, 0)),
            pl.BlockSpec(Wp.shape, lambda i: (0, 0)),
            pl.BlockSpec(A1.shape, lambda i: (0, 0)),
            pl.BlockSpec(b1.shape, lambda i: (0, 0)),
        ],
        out_specs=[pl.BlockSpec((1, k, s), lambda i: (i, 0, 0)),
                   pl.BlockSpec((1, n, 2 * h1), lambda i: (i, 0, 0)),
                   pl.BlockSpec((1, s, h1), lambda i: (i, 0, 0))],
        out_shape=[jax.ShapeDtypeStruct((b, k, s), jnp.int32),
                   jax.ShapeDtypeStruct((b, n, 2 * h1), jnp.float32),
                   jax.ShapeDtypeStruct((b, s, h1), jnp.float32)],
    )(pt, cxyz, cn, Wp, A1, b1)


def _sc_gather(table, idx):
    rows, d = table.shape
    nrows = idx.shape[0]
    info = plsc.get_sparse_core_info()
    nw = info.num_cores * info.num_subcores
    per_w = nrows // nw
    ch = 256
    assert per_w % ch == 0 and nrows % (8 * nw) == 0
    mesh = plsc.VectorSubcoreMesh(core_axis_name="c", subcore_axis_name="s")

    @functools.partial(
        pl.kernel, mesh=mesh,
        out_type=jax.ShapeDtypeStruct((nrows, d), jnp.float32),
        scratch_types=[
            pltpu.VMEM((per_w,), jnp.int32),
            pltpu.VMEM((ch, d), jnp.float32),
            pltpu.SemaphoreType.DMA,
        ],
    )
    def gk(table_hbm, idx_hbm, out_hbm, idx_v, rows_v, sem):
        wid = jax.lax.axis_index("s") * info.num_cores + jax.lax.axis_index("c")
        base = wid * per_w
        pltpu.sync_copy(idx_hbm.at[pl.ds(base, per_w)], idx_v)
        for i in range(per_w // ch):
            pltpu.async_copy(
                table_hbm.at[idx_v.at[pl.ds(i * ch, ch)]], rows_v, sem).wait()
            pltpu.sync_copy(rows_v, out_hbm.at[pl.ds(base + i * ch, ch)])

    return gk(table, idx)


def _post1_kernel(g_ref, beta_ref, W2_ref, out_ref, *, k):
    beta = beta_ref[0]
    s = beta.shape[0]
    acc = None
    for j in range(k):
        Hh = jax.nn.relu(g_ref[0, j * s:(j + 1) * s, 0:beta.shape[1]] + beta)
        O = jax.lax.dot_general(Hh, W2_ref[...], (((1,), (1,)), ((), ())),
                                precision=jax.lax.Precision.HIGHEST)
        acc = O if acc is None else jnp.maximum(acc, O)
    out_ref[0] = acc


def _post1(g, beta, W2, k):
    b, s, h1 = beta.shape
    h2 = W2.shape[0]
    return pl.pallas_call(
        functools.partial(_post1_kernel, k=k),
        grid=(b,),
        in_specs=[
            pl.BlockSpec((1, k * s, g.shape[2]), lambda i: (i, 0, 0)),
            pl.BlockSpec((1, s, h1), lambda i: (i, 0, 0)),
            pl.BlockSpec(W2.shape, lambda i: (0, 0)),
        ],
        out_specs=pl.BlockSpec((1, s, h2), lambda i: (i, 0, 0)),
        out_shape=jax.ShapeDtypeStruct((b, s, h2), jnp.float32),
    )(g, beta, W2)


def _sa2_kernel(pt_ref, cxyz_ref, cn_ref, feat_ref, A1_ref, A2_ref, b1_ref,
                W2_ref, out_ref, *, k):
    Pt = pt_ref[0]
    C = cxyz_ref[0]
    feat = feat_ref[0]
    sq = _sq_dists_t(Pt, C, cn_ref[0])
    F = (jax.lax.dot_general(A1_ref[...], Pt, (((1,), (1,)), ((), ())),
                             precision=jax.lax.Precision.HIGHEST)
         + jax.lax.dot_general(A2_ref[...], feat, (((1,), (1,)), ((), ())),
                               precision=jax.lax.Precision.HIGHEST))
    beta = b1_ref[...] - jax.lax.dot_general(
        C, A1_ref[...], (((0,), (1,)), ((), ())),
        precision=jax.lax.Precision.HIGHEST)
    out_ref[0] = _knn_mlp_pool_t(sq, F, beta, W2_ref[...], k)


def _sa2(pt, cxyz, cn, feat, A1, A2, b1, W2, k):
    b, n, _ = pt.shape
    s = cxyz.shape[2]
    cf = feat.shape[2]
    h2 = W2.shape[0]
    return pl.pallas_call(
        functools.partial(_sa2_kernel, k=k),
        grid=(b,),
        in_specs=[
            pl.BlockSpec((1, n, 3), lambda i: (i, 0, 0)),
            pl.BlockSpec((1, 3, s), lambda i: (i, 0, 0)),
            pl.BlockSpec((1, 1, s), lambda i: (i, 0, 0)),
            pl.BlockSpec((1, n, cf), lambda i: (i, 0, 0)),
            pl.BlockSpec(A1.shape, lambda i: (0, 0)),
            pl.BlockSpec(A2.shape, lambda i: (0, 0)),
            pl.BlockSpec(b1.shape, lambda i: (0, 0)),
            pl.BlockSpec(W2.shape, lambda i: (0, 0)),
        ],
        out_specs=pl.BlockSpec((1, s, h2), lambda i: (i, 0, 0)),
        out_shape=jax.ShapeDtypeStruct((b, s, h2), jnp.float32),
    )(pt, cxyz, cn, feat, A1, A2, b1, W2)


def _tail_kernel(xyz_ref, feat_ref, W3x_ref, W3f_ref, b3a_ref, W3b_ref,
                 b3b_ref, Wf1_ref, bn1_ref, W1s_ref, Wf2_ref, bn2_ref,
                 W2s_ref, logit_ref):
    xyz = xyz_ref[0]
    feat = feat_ref[0]
    h = jax.nn.relu(
        jax.lax.dot_general(W3x_ref[...], xyz, (((1,), (0,)), ((), ())),
                            precision=jax.lax.Precision.HIGHEST)
        + jax.lax.dot_general(W3f_ref[...], feat, (((1,), (1,)), ((), ())),
                              precision=jax.lax.Precision.HIGHEST)
        + b3a_ref[...])
    o = jax.lax.dot_general(W3b_ref[...], h, (((1,), (0,)), ((), ())),
                            precision=jax.lax.Precision.HIGHEST)
    o = o + b3b_ref[...]
    x = jnp.max(o, axis=1, keepdims=True)
    rs = jnp.sqrt(1.0 + EPS)
    y = jax.lax.dot_general(Wf1_ref[...], x, (((1,), (0,)), ((), ())),
                            precision=jax.lax.Precision.HIGHEST)
    y = W1s_ref[...] * (y + bn1_ref[..., 0:1]) / rs + bn1_ref[..., 1:2]
    y = jax.nn.relu(y)
    z = jax.lax.dot_general(Wf2_ref[...], y, (((1,), (0,)), ((), ())),
                            precision=jax.lax.Precision.HIGHEST)
    z = W2s_ref[...] * (z + bn2_ref[..., 0:1]) / rs + bn2_ref[..., 1:2]
    logit_ref[0] = jax.nn.relu(z)


def _tail(xyz, feat, W3a, b3a, W3b, b3b, Wf1, bf1, gf1, bef1, Wf2, bf2, gf2,
          bef2):
    b, _, s = xyz.shape
    W3x = W3a[:, :3]
    W3f = W3a[:, 3:]
    b3ac = b3a[:, None]
    b3bc = b3b[:, None]
    bn1 = jnp.stack([bf1, bef1], axis=1)
    bn2 = jnp.stack([bf2, bef2], axis=1)
    g1c = gf1[:, None]
    g2c = gf2[:, None]
    full = lambda a: pl.BlockSpec(a.shape, lambda i: tuple(0 for _ in a.shape))
    out = pl.pallas_call(
        _tail_kernel,
        grid=(b,),
        in_specs=[
            pl.BlockSpec((1, 3, s), lambda i: (i, 0, 0)),
            pl.BlockSpec((1, s, feat.shape[2]), lambda i: (i, 0, 0)),
            full(W3x), full(W3f), full(b3ac), full(W3b), full(b3bc),
            full(Wf1), full(bn1), full(g1c), full(Wf2), full(bn2), full(g2c),
        ],
        out_specs=pl.BlockSpec((1, 128, 1), lambda i: (i, 0, 0)),
        out_shape=jax.ShapeDtypeStruct((b, 128, 1), jnp.float32),
    )(xyz, feat, W3x, W3f, b3ac, W3b, b3bc, Wf1, bn1, g1c, Wf2, bn2, g2c)
    return out[:, :, 0]


def kernel(point_cloud, W1a, b1a, W1b, b1b, W2a, b2a, W2b, b2b, W3a, b3a,
           W3b, b3b, Wf1, bf1, gf1, bef1, Wf2, bf2, gf2, bef2):
    pc = point_cloud
    b, _, n = pc.shape
    pc_t = jnp.transpose(pc, (0, 2, 1))
    _, l1_xyz, cn1 = _fps(pc, 512)
    Wp1 = W1a[:, :3] + W1a[:, 3:]
    idx1, ft1, beta1 = _sel1(pc_t, l1_xyz, cn1[:, None, :], Wp1, W1a[:, :3],
                             b1a[None, :], 12)
    g1 = _sc_gather(ft1.reshape(b * n, ft1.shape[2]), idx1.reshape(-1))
    p1 = _post1(g1.reshape(b, -1, ft1.shape[2]), beta1, W1b, 12)
    l1_t = jnp.transpose(l1_xyz, (0, 2, 1))
    _, l2_xyz, cn2 = _fps(l1_xyz, 128)
    p2 = _sa2(l1_t, l2_xyz, cn2[:, None, :], p1, W2a[:, :3], W2a[:, 3:],
              b2a[None, :], W2b, 8)
    logit = _tail(l2_xyz, p2, W3a, b3a, W3b, b3b,
                  Wf1, bf1, gf1, bef1, Wf2, bf2, gf2, bef2)
    l21_points = jnp.transpose(p2, (0, 2, 1))
    return (logit, l21_points)
